# Initial kernel scaffold; baseline (speedup 1.0000x reference)
import jax
import jax.numpy as jnp
from jax.experimental import pallas as pl


def kernel(x, edge_index, W_l1, W_r1, b1, gamma1, beta1, W_l2, W_r2, b2):
    raise NotImplementedError("write your pallas kernel here")


# speedup vs baseline: 5.5623x; 5.5623x over previous
import jax
import jax.numpy as jnp
from jax import lax
from jax.experimental import pallas as pl
from jax.experimental.pallas import tpu as pltpu
from jax.experimental.pallas import tpu_sc as plsc

N_NODES = 10000
D = 128
E = 320000

_NC = 2
_NS = 16
_NW = _NC * _NS
_K = 80
_EPW = E // _NW
_CPW = _EPW // _K
_NPAD = 10240
_RPS = _NPAD // _NS


def _sc_agg_body(x_hbm, src_hbm, dst_hbm, zrows_hbm,
                 sum_hbm, deg_hbm,
                 idx_s, idx_d, rows, deg_t, sem, agg_s):
  cid = lax.axis_index("c")
  sid = lax.axis_index("s")
  wid = cid * _NS + sid

  r0 = pl.multiple_of(sid * _RPS, 8)
  pltpu.sync_copy(zrows_hbm, rows)
  for j in range(_RPS // _K):
    pltpu.sync_copy(rows, agg_s.at[pl.ds(r0 + j * _K, _K)])

  zeros16 = jnp.zeros((16,), jnp.float32)

  def zero_deg(i, carry):
    deg_t[pl.ds(i * 16, 16)] = zeros16
    return carry

  lax.fori_loop(0, _NPAD // 16, zero_deg, 0)
  plsc.subcore_barrier()

  e0 = wid * _EPW
  ones16 = jnp.ones((16,), jnp.float32)

  def chunk(i, carry):
    base = pl.multiple_of(e0 + i * _K, 8)
    pltpu.sync_copy(src_hbm.at[pl.ds(base, _K)], idx_s)
    pltpu.sync_copy(dst_hbm.at[pl.ds(base, _K)], idx_d)
    pltpu.async_copy(x_hbm.at[idx_s], rows, sem).wait()
    pltpu.sync_copy(rows, agg_s.at[idx_d], add=True)
    for j in range(_K // 16):
      dv = idx_d[pl.ds(j * 16, 16)]
      plsc.addupdate_scatter(deg_t, [dv], ones16)
    return carry

  lax.fori_loop(0, _CPW, chunk, 0)
  plsc.subcore_barrier()

  o0 = pl.multiple_of(cid * _NPAD + sid * _RPS, 8)
  for j in range(_RPS // _K):
    pltpu.sync_copy(agg_s.at[pl.ds(r0 + j * _K, _K)], rows)
    pltpu.sync_copy(rows, sum_hbm.at[pl.ds(o0 + j * _K, _K)])
  t0 = pl.multiple_of(wid * _NPAD, 8)
  pltpu.sync_copy(deg_t, deg_hbm.at[pl.ds(t0, _NPAD)])


_sc_agg = pl.kernel(
    _sc_agg_body,
    out_type=[
        jax.ShapeDtypeStruct((_NC * _NPAD, D), jnp.float32),
        jax.ShapeDtypeStruct((_NW * _NPAD,), jnp.float32),
    ],
    mesh=plsc.VectorSubcoreMesh(core_axis_name="c", subcore_axis_name="s"),
    compiler_params=pltpu.CompilerParams(needs_layout_passes=False),
    scratch_types=[
        pltpu.VMEM((_K,), jnp.int32),
        pltpu.VMEM((_K,), jnp.int32),
        pltpu.VMEM((_K, D), jnp.float32),
        pltpu.VMEM((_NPAD,), jnp.float32),
        pltpu.SemaphoreType.DMA,
        pltpu.VMEM_SHARED((_NPAD, D), jnp.float32),
    ],
)


def _dotT(a, w):
  return lax.dot_general(a, w, (((1,), (1,)), ((), ())),
                         preferred_element_type=jnp.float32)


def _mean_from_partials(s_ref, d_ref):
  deg = jnp.sum(d_ref[...], axis=1, keepdims=True)[:N_NODES]
  deg = jnp.maximum(deg, 1.0)
  return (s_ref[:N_NODES] + s_ref[_NPAD:_NPAD + N_NODES]) / deg


def _tc_layer1_body(x_ref, s_ref, d_ref, wl_ref, wr_ref, b_ref, g_ref, be_ref,
                    h_ref):
  mean = _mean_from_partials(s_ref, d_ref)
  t = _dotT(mean, wl_ref[...]) + _dotT(x_ref[...], wr_ref[...]) + b_ref[...]
  mu = jnp.mean(t, axis=0, keepdims=True)
  var = jnp.mean((t - mu) * (t - mu), axis=0, keepdims=True)
  h = g_ref[...] * (t - mu) * lax.rsqrt(var + 1e-5) + be_ref[...]
  h_ref[...] = jnp.maximum(h, 0.0)


def _tc_layer2_body(h_ref, s_ref, d_ref, wl_ref, wr_ref, b_ref, o_ref):
  mean = _mean_from_partials(s_ref, d_ref)
  o_ref[...] = _dotT(mean, wl_ref[...]) + _dotT(h_ref[...], wr_ref[...]) \
      + b_ref[...]


_tc_layer1 = pl.pallas_call(
    _tc_layer1_body,
    out_shape=jax.ShapeDtypeStruct((N_NODES, D), jnp.float32),
)

_tc_layer2 = pl.pallas_call(
    _tc_layer2_body,
    out_shape=jax.ShapeDtypeStruct((N_NODES, D), jnp.float32),
)


@jax.jit
def kernel(x, edge_index, W_l1, W_r1, b1, gamma1, beta1, W_l2, W_r2, b2):
  src = edge_index[0].astype(jnp.int32)
  dst = edge_index[1].astype(jnp.int32)
  zrows = jnp.zeros((_K, D), jnp.float32)

  sum1, degp = _sc_agg(x, src, dst, zrows)
  degT = degp.reshape(_NW, _NPAD).T

  h = _tc_layer1(x, sum1, degT, W_l1, W_r1, b1.reshape(1, D),
                 gamma1.reshape(1, D), beta1.reshape(1, D))
  sum2, _ = _sc_agg(h, src, dst, zrows)
  out = _tc_layer2(h, sum2, degT, W_l2, W_r2, b2.reshape(1, D))
  return out


# speedup vs baseline: 8.7754x; 1.5777x over previous
import jax
import jax.numpy as jnp
from jax import lax
from jax.experimental import pallas as pl
from jax.experimental.pallas import tpu as pltpu
from jax.experimental.pallas import tpu_sc as plsc

N_NODES = 10000
D = 128
E = 320000

_NC = 2
_NS = 16
_NW = _NC * _NS
_K = 80
_EPW = E // _NW
_CPW = _EPW // _K
_NPAD = 10240
_RPS = _NPAD // _NS


def _sc_agg_body(x_hbm, src_hbm, dst_hbm, zrows_hbm,
                 sum_hbm, deg_hbm,
                 idx_s0, idx_d0, rows0, idx_s1, idx_d1, rows1,
                 sem0, sem1, deg_t, agg_s):
  cid = lax.axis_index("c")
  sid = lax.axis_index("s")
  wid = cid * _NS + sid

  r0 = pl.multiple_of(sid * _RPS, 8)
  pltpu.sync_copy(zrows_hbm, rows0)
  for j in range(_RPS // _K):
    pltpu.sync_copy(rows0, agg_s.at[pl.ds(r0 + j * _K, _K)])

  zeros16 = jnp.zeros((16,), jnp.float32)

  def zero_deg(i, carry):
    deg_t[pl.ds(i * 16, 16)] = zeros16
    return carry

  lax.fori_loop(0, _NPAD // 16, zero_deg, 0)
  plsc.subcore_barrier()

  e0 = wid * _EPW
  ones16 = jnp.ones((16,), jnp.float32)

  def load_idx(c, idx_s, idx_d):
    base = pl.multiple_of(e0 + c * _K, 8)
    pltpu.sync_copy(src_hbm.at[pl.ds(base, _K)], idx_s)
    pltpu.sync_copy(dst_hbm.at[pl.ds(base, _K)], idx_d)

  def process(idx_d, rows):
    pltpu.sync_copy(rows, agg_s.at[idx_d], add=True)
    for j in range(_K // 16):
      dv = idx_d[pl.ds(j * 16, 16)]
      plsc.addupdate_scatter(deg_t, [dv], ones16)

  load_idx(0, idx_s0, idx_d0)
  pltpu.async_copy(x_hbm.at[idx_s0], rows0, sem0)

  def pair(g2, carry):
    c = g2 * 2
    load_idx(c + 1, idx_s1, idx_d1)
    g1 = pltpu.async_copy(x_hbm.at[idx_s1], rows1, sem1)
    pltpu.make_async_copy(x_hbm.at[idx_s0], rows0, sem0).wait()
    process(idx_d0, rows0)
    load_idx(c + 2, idx_s0, idx_d0)
    pltpu.async_copy(x_hbm.at[idx_s0], rows0, sem0)
    g1.wait()
    process(idx_d1, rows1)
    return carry

  lax.fori_loop(0, (_CPW - 1) // 2, pair, 0)
  pltpu.make_async_copy(x_hbm.at[idx_s0], rows0, sem0).wait()
  process(idx_d0, rows0)
  plsc.subcore_barrier()

  o0 = pl.multiple_of(cid * _NPAD + sid * _RPS, 8)
  for j in range(_RPS // _K):
    pltpu.sync_copy(agg_s.at[pl.ds(r0 + j * _K, _K)], rows0)
    pltpu.sync_copy(rows0, sum_hbm.at[pl.ds(o0 + j * _K, _K)])
  t0 = pl.multiple_of(wid * _NPAD, 8)
  pltpu.sync_copy(deg_t, deg_hbm.at[pl.ds(t0, _NPAD)])


_sc_agg = pl.kernel(
    _sc_agg_body,
    out_type=[
        jax.ShapeDtypeStruct((_NC * _NPAD, D), jnp.float32),
        jax.ShapeDtypeStruct((_NW * _NPAD,), jnp.float32),
    ],
    mesh=plsc.VectorSubcoreMesh(core_axis_name="c", subcore_axis_name="s"),
    compiler_params=pltpu.CompilerParams(needs_layout_passes=False),
    scratch_types=[
        pltpu.VMEM((_K,), jnp.int32),
        pltpu.VMEM((_K,), jnp.int32),
        pltpu.VMEM((_K, D), jnp.float32),
        pltpu.VMEM((_K,), jnp.int32),
        pltpu.VMEM((_K,), jnp.int32),
        pltpu.VMEM((_K, D), jnp.float32),
        pltpu.SemaphoreType.DMA,
        pltpu.SemaphoreType.DMA,
        pltpu.VMEM((_NPAD,), jnp.float32),
        pltpu.VMEM_SHARED((_NPAD, D), jnp.float32),
    ],
)


def _dotT(a, w):
  return lax.dot_general(a, w, (((1,), (1,)), ((), ())),
                         preferred_element_type=jnp.float32)


def _mean_from_partials(s_ref, d_ref):
  deg = jnp.sum(d_ref[...], axis=1, keepdims=True)[:N_NODES]
  deg = jnp.maximum(deg, 1.0)
  return (s_ref[:N_NODES] + s_ref[_NPAD:_NPAD + N_NODES]) / deg


def _tc_layer1_body(x_ref, s_ref, d_ref, wl_ref, wr_ref, b_ref, g_ref, be_ref,
                    h_ref):
  mean = _mean_from_partials(s_ref, d_ref)
  t = _dotT(mean, wl_ref[...]) + _dotT(x_ref[...], wr_ref[...]) + b_ref[...]
  mu = jnp.mean(t, axis=0, keepdims=True)
  var = jnp.mean((t - mu) * (t - mu), axis=0, keepdims=True)
  h = g_ref[...] * (t - mu) * lax.rsqrt(var + 1e-5) + be_ref[...]
  h_ref[...] = jnp.maximum(h, 0.0)


def _tc_layer2_body(h_ref, s_ref, d_ref, wl_ref, wr_ref, b_ref, o_ref):
  mean = _mean_from_partials(s_ref, d_ref)
  o_ref[...] = _dotT(mean, wl_ref[...]) + _dotT(h_ref[...], wr_ref[...]) \
      + b_ref[...]


_tc_layer1 = pl.pallas_call(
    _tc_layer1_body,
    out_shape=jax.ShapeDtypeStruct((N_NODES, D), jnp.float32),
)

_tc_layer2 = pl.pallas_call(
    _tc_layer2_body,
    out_shape=jax.ShapeDtypeStruct((N_NODES, D), jnp.float32),
)


@jax.jit
def kernel(x, edge_index, W_l1, W_r1, b1, gamma1, beta1, W_l2, W_r2, b2):
  src = edge_index[0].astype(jnp.int32)
  dst = edge_index[1].astype(jnp.int32)
  zrows = jnp.zeros((_K, D), jnp.float32)

  sum1, degp = _sc_agg(x, src, dst, zrows)
  degT = degp.reshape(_NW, _NPAD).T

  h = _tc_layer1(x, sum1, degT, W_l1, W_r1, b1.reshape(1, D),
                 gamma1.reshape(1, D), beta1.reshape(1, D))
  sum2, _ = _sc_agg(h, src, dst, zrows)
  out = _tc_layer2(h, sum2, degT, W_l2, W_r2, b2.reshape(1, D))
  return out


# speedup vs baseline: 10.6599x; 1.2148x over previous
import jax
import jax.numpy as jnp
from jax import lax
from jax.experimental import pallas as pl
from jax.experimental.pallas import tpu as pltpu
from jax.experimental.pallas import tpu_sc as plsc

N_NODES = 10000
D = 128
E = 320000

_NC = 2
_NS = 16
_NW = _NC * _NS
_K = 80
_EPW = E // _NW
_CPW = _EPW // _K
_PLD = 1000
_NPAD = 10240
_RPS = _NPAD // _NS


def _sc_agg_body(x_hbm, src_hbm, dst_hbm, zrows_hbm,
                 sum_hbm, deg_hbm,
                 idx_s0, idx_d0, rows0, idx_s1, idx_d1, rows1,
                 sem0, sem1, semi0, semi1, deg_t, agg_s):
  cid = lax.axis_index("c")
  sid = lax.axis_index("s")
  wid = cid * _NS + sid

  r0 = pl.multiple_of(sid * _RPS, 8)
  pltpu.sync_copy(zrows_hbm, rows0)
  for j in range(_RPS // _K):
    pltpu.sync_copy(rows0, agg_s.at[pl.ds(r0 + j * _K, _K)])

  zeros16 = jnp.zeros((16,), jnp.float32)

  def zero_deg(i, carry):
    deg_t[pl.ds(i * 16, 16)] = zeros16
    return carry

  lax.fori_loop(0, _NPAD // 16, zero_deg, 0)
  plsc.subcore_barrier()

  e0 = wid * _EPW
  ones16 = jnp.ones((16,), jnp.float32)

  def start_idx(c, idx_s, idx_d, semi):
    base = pl.multiple_of(e0 + c * _K, 8)
    pltpu.async_copy(src_hbm.at[pl.ds(base, _K)], idx_s, semi)
    pltpu.async_copy(dst_hbm.at[pl.ds(base, _K)], idx_d, semi)

  def wait_idx(c, idx_s, idx_d, semi):
    base = pl.multiple_of(e0 + c * _K, 8)
    pltpu.make_async_copy(src_hbm.at[pl.ds(base, _K)], idx_s, semi).wait()
    pltpu.make_async_copy(dst_hbm.at[pl.ds(base, _K)], idx_d, semi).wait()

  def start_gather(idx_s, rows, sem):
    return pltpu.async_copy(x_hbm.at[idx_s], rows, sem)

  def wait_gather(idx_s, rows, sem):
    pltpu.make_async_copy(x_hbm.at[idx_s], rows, sem).wait()

  def process(idx_d, rows):
    pltpu.sync_copy(rows, agg_s.at[idx_d], add=True)
    for j in range(_K // 16):
      dv = idx_d[pl.ds(j * 16, 16)]
      plsc.addupdate_scatter(deg_t, [dv], ones16)

  start_idx(0, idx_s0, idx_d0, semi0)
  wait_idx(0, idx_s0, idx_d0, semi0)
  start_gather(idx_s0, rows0, sem0)
  start_idx(1, idx_s1, idx_d1, semi1)

  def pair(g2, carry):
    c = g2 * 2
    wait_idx(c + 1, idx_s1, idx_d1, semi1)
    start_gather(idx_s1, rows1, sem1)
    wait_gather(idx_s0, rows0, sem0)
    process(idx_d0, rows0)
    start_idx(c + 2, idx_s0, idx_d0, semi0)
    wait_idx(c + 2, idx_s0, idx_d0, semi0)
    start_gather(idx_s0, rows0, sem0)
    wait_gather(idx_s1, rows1, sem1)
    process(idx_d1, rows1)
    start_idx(c + 3, idx_s1, idx_d1, semi1)
    return carry

  lax.fori_loop(0, (_CPW - 1) // 2, pair, 0)
  wait_idx(_CPW, idx_s1, idx_d1, semi1)
  wait_gather(idx_s0, rows0, sem0)
  process(idx_d0, rows0)
  plsc.subcore_barrier()

  o0 = pl.multiple_of(cid * _NPAD + sid * _RPS, 8)
  for j in range(_RPS // _K):
    pltpu.sync_copy(agg_s.at[pl.ds(r0 + j * _K, _K)], rows0)
    pltpu.sync_copy(rows0, sum_hbm.at[pl.ds(o0 + j * _K, _K)])
  t0 = pl.multiple_of(wid * _NPAD, 8)
  pltpu.sync_copy(deg_t, deg_hbm.at[pl.ds(t0, _NPAD)])


_sc_agg = pl.kernel(
    _sc_agg_body,
    out_type=[
        jax.ShapeDtypeStruct((_NC * _NPAD, D), jnp.float32),
        jax.ShapeDtypeStruct((_NW * _NPAD,), jnp.float32),
    ],
    mesh=plsc.VectorSubcoreMesh(core_axis_name="c", subcore_axis_name="s"),
    compiler_params=pltpu.CompilerParams(needs_layout_passes=False),
    scratch_types=[
        pltpu.VMEM((_K,), jnp.int32),
        pltpu.VMEM((_K,), jnp.int32),
        pltpu.VMEM((_K, D), jnp.float32),
        pltpu.VMEM((_K,), jnp.int32),
        pltpu.VMEM((_K,), jnp.int32),
        pltpu.VMEM((_K, D), jnp.float32),
        pltpu.SemaphoreType.DMA,
        pltpu.SemaphoreType.DMA,
        pltpu.SemaphoreType.DMA,
        pltpu.SemaphoreType.DMA,
        pltpu.VMEM((_NPAD,), jnp.float32),
        pltpu.VMEM_SHARED((_NPAD, D), jnp.float32),
    ],
)


def _dotT(a, w):
  return lax.dot_general(a, w, (((1,), (1,)), ((), ())),
                         preferred_element_type=jnp.float32)


def _mean_from_partials(s_ref, d_ref):
  deg = jnp.sum(d_ref[...], axis=1, keepdims=True)[:N_NODES]
  deg = jnp.maximum(deg, 1.0)
  return (s_ref[:N_NODES] + s_ref[_NPAD:_NPAD + N_NODES]) / deg


def _tc_layer1_body(x_ref, s_ref, d_ref, wl_ref, wr_ref, b_ref, g_ref, be_ref,
                    h_ref):
  mean = _mean_from_partials(s_ref, d_ref)
  t = _dotT(mean, wl_ref[...]) + _dotT(x_ref[...], wr_ref[...]) + b_ref[...]
  mu = jnp.mean(t, axis=0, keepdims=True)
  var = jnp.mean((t - mu) * (t - mu), axis=0, keepdims=True)
  h = g_ref[...] * (t - mu) * lax.rsqrt(var + 1e-5) + be_ref[...]
  h_ref[...] = jnp.maximum(h, 0.0)


def _tc_layer2_body(h_ref, s_ref, d_ref, wl_ref, wr_ref, b_ref, o_ref):
  mean = _mean_from_partials(s_ref, d_ref)
  o_ref[...] = _dotT(mean, wl_ref[...]) + _dotT(h_ref[...], wr_ref[...]) \
      + b_ref[...]


_tc_layer1 = pl.pallas_call(
    _tc_layer1_body,
    out_shape=jax.ShapeDtypeStruct((N_NODES, D), jnp.float32),
)

_tc_layer2 = pl.pallas_call(
    _tc_layer2_body,
    out_shape=jax.ShapeDtypeStruct((N_NODES, D), jnp.float32),
)


@jax.jit
def kernel(x, edge_index, W_l1, W_r1, b1, gamma1, beta1, W_l2, W_r2, b2):
  pad = jnp.zeros((2 * _K,), jnp.int32)
  src = jnp.concatenate([edge_index[0].astype(jnp.int32), pad])
  dst = jnp.concatenate([edge_index[1].astype(jnp.int32), pad])
  zrows = jnp.zeros((_K, D), jnp.float32)

  sum1, degp = _sc_agg(x, src, dst, zrows)
  degT = degp.reshape(_NW, _NPAD).T

  h = _tc_layer1(x, sum1, degT, W_l1, W_r1, b1.reshape(1, D),
                 gamma1.reshape(1, D), beta1.reshape(1, D))
  sum2, _ = _sc_agg(h, src, dst, zrows)
  out = _tc_layer2(h, sum2, degT, W_l2, W_r2, b2.reshape(1, D))
  return out
